# baseline (speedup 1.0000x reference)
import jax
import jax.numpy as jnp
from jax import lax
from jax.experimental import pallas as pl
from jax.experimental.pallas import tpu as pltpu
from jax.experimental.pallas import tpu_sc as plsc

N = 10000
DEG = 16
D = 256
SIM_D = 128
K = 10
N_LAYERS = 3
N_CLASSES = 2
EPS = 1e-07

NW = 32
NP = 10240
SA0 = 608
SA1 = 32
SOFF = 16 * SA0
GA0 = 576
GA1 = 64
GOFF = 16 * GA0
CB = 8
CBL = 8
NBUF = 4
SW = 8
BR = 256
NB = NP // BR

_mesh = plsc.VectorSubcoreMesh(
    core_axis_name="c", subcore_axis_name="s", num_cores=2, num_subcores=16)


def _proj_body(x_ref, w_ref, b_ref, feat_ref, qn_ref):
    f = jnp.dot(x_ref[...], w_ref[...],
                preferred_element_type=jnp.float32) + b_ref[...]
    feat_ref[...] = f
    qn_ref[...] = jnp.sum(f * f, axis=1, keepdims=True)


def _project(x_pad, w_sim, b_sim):
    return pl.pallas_call(
        _proj_body,
        grid=(NB,),
        in_specs=[
            pl.BlockSpec((BR, D), lambda j: (j, 0)),
            pl.BlockSpec((D, SIM_D), lambda j: (0, 0)),
            pl.BlockSpec((1, SIM_D), lambda j: (0, 0)),
        ],
        out_specs=[
            pl.BlockSpec((BR, SIM_D), lambda j: (j, 0)),
            pl.BlockSpec((BR, 1), lambda j: (j, 0)),
        ],
        out_shape=[
            jax.ShapeDtypeStruct((NP, SIM_D), jnp.float32),
            jax.ShapeDtypeStruct((NP, 1), jnp.float32),
        ],
    )(x_pad, w_sim, b_sim)


def _tree_sum(vals):
    while len(vals) > 1:
        nxt = [vals[i] + vals[i + 1] for i in range(0, len(vals) - 1, 2)]
        if len(vals) % 2:
            nxt.append(vals[-1])
        vals = nxt
    return vals[0]


def _sim_body(feat_hbm, qn_hbm, src_hbm, xx_hbm, ssrc_hbm,
              qn_v, src_v, dstf_v, srcf_v, xx_v, ssrc_v,
              sem_g0, sem_g1, sem_g2, sem_g3, sem_s):
    cid = lax.axis_index("c")
    sid = lax.axis_index("s")
    sems = (sem_g0, sem_g1, sem_g2, sem_g3)
    base = jnp.where(cid == 0, sid * SA0, SOFF + sid * SA1)
    NC = jnp.where(cid == 0, SA0 // CB, SA1 // CB)

    def _make_descs(fsrc):
        def _descs(cb, par):
            idx = src_v.at[pl.ds(cb * CB * DEG, CB * DEG)]
            g = pltpu.make_async_copy(
                fsrc.at[idx], srcf_v.at[par], sems[par])
            d = pltpu.make_async_copy(
                fsrc.at[pl.ds(base + cb * CB, CB)], dstf_v.at[par],
                sems[par])
            return g, d
        return _descs

    def _xx_desc(cb):
        return pltpu.make_async_copy(
            xx_v.at[pl.ds(cb * CB * DEG, CB * DEG)],
            xx_hbm.at[pl.ds((base + cb * CB) * DEG, CB * DEG)], sem_s)

    def _ssrc_desc(cb):
        return pltpu.make_async_copy(
            ssrc_v.at[pl.ds(cb * CB * DEG, CB * DEG)],
            ssrc_hbm.at[pl.ds((base + cb * CB) * DEG, CB * DEG)], sem_s)

    def _chunk_body(cb, _descs, _fire):
        par_d = lax.rem(cb, NBUF)
        lb = cb * CB
        for par in range(NBUF):
            @pl.when(par_d == par)
            def _wait():
                g, d = _descs(cb, par)
                g.wait()
                d.wait()

        @pl.loop(0, CB)
        def _node(i):
            src_ids = src_v[pl.ds((lb + i) * DEG, DEG)]
            qs = plsc.load_gather(
                qn_v, [lax.shift_right_logical(src_ids, 7),
                       lax.bitwise_and(src_ids, 127)])
            dvec = [dstf_v[par_d, i, pl.ds(16 * k, 16)]
                    for k in range(SIM_D // 16)]
            qd = jnp.sum(_tree_sum([dv * dv for dv in dvec]))
            lane = lax.broadcasted_iota(jnp.int32, (DEG,), 0)
            dotv = jnp.zeros((DEG,), jnp.float32)
            for e in range(DEG):
                r = i * DEG + e
                prods = [srcf_v[par_d, r, pl.ds(16 * k, 16)] * dvec[k]
                         for k in range(SIM_D // 16)]
                dotv = jnp.where(lane == e,
                                 jnp.sum(_tree_sum(prods)), dotv)
            sqd = qd + qs - 2.0 * dotv
            xx = 1.0 + 2.0 * sqd / ((1.0 - qd) * (1.0 - qs)) + EPS
            sk, sv = plsc.sort_key_val(xx, src_ids, descending=True)
            xx_v[pl.ds((lb + i) * DEG, DEG)] = sk
            ssrc_v[pl.ds((lb + i) * DEG, DEG)] = sv

        _xx_desc(cb).start()
        _ssrc_desc(cb).start()

        @pl.when(cb >= SW)
        def _throttle_stores():
            _xx_desc(cb - SW).wait()
            _ssrc_desc(cb - SW).wait()

        for par in range(NBUF):
            @pl.when((par_d == par) & (cb + NBUF - 1 < NC))
            def _prefetch():
                _fire(cb + NBUF - 1, (par + NBUF - 1) % NBUF)

    pltpu.sync_copy(qn_hbm, qn_v)
    pltpu.sync_copy(src_hbm.at[pl.ds(base * DEG, SA0 * DEG)], src_v)

    def _pipeline(fsrc):
        _descs = _make_descs(fsrc)

        def _fire(cb, par):
            g, d = _descs(cb, par)
            g.start()
            d.start()

        for p in range(NBUF - 1):
            _fire(p, p)

        @pl.loop(0, NC)
        def _chunks(cb):
            _chunk_body(cb, _descs, _fire)

        @pl.loop(jnp.maximum(NC - SW, 0), NC)
        def _drain(cb):
            _xx_desc(cb).wait()
            _ssrc_desc(cb).wait()

    _pipeline(feat_hbm)


def _sim_topk(feat, qn, src_pad):
    call = pl.kernel(
        _sim_body,
        out_type=[
            jax.ShapeDtypeStruct((NP * DEG,), jnp.float32),
            jax.ShapeDtypeStruct((NP * DEG,), jnp.int32),
        ],
        mesh=_mesh,
        scratch_types=[
            pltpu.VMEM((NP // SIM_D, SIM_D), jnp.float32),
            pltpu.VMEM((SA0 * DEG,), jnp.int32),
            pltpu.VMEM((NBUF, CB, SIM_D), jnp.float32),
            pltpu.VMEM((NBUF, CB * DEG, SIM_D), jnp.float32),
            pltpu.VMEM((SA0 * DEG,), jnp.float32),
            pltpu.VMEM((SA0 * DEG,), jnp.int32),
            pltpu.SemaphoreType.DMA,
            pltpu.SemaphoreType.DMA,
            pltpu.SemaphoreType.DMA,
            pltpu.SemaphoreType.DMA,
            pltpu.SemaphoreType.DMA,
        ],
        compiler_params=pltpu.CompilerParams(needs_layout_passes=False),
    )
    return call(feat, qn, src_pad)


def _gmean_body(h_hbm, sel_hbm, out_hbm, idx_v, rows_v, out_v,
                sem_g0, sem_g1, sem_g2, sem_g3,
                sem_s0, sem_s1, sem_s2, sem_s3):
    cid = lax.axis_index("c")
    sid = lax.axis_index("s")
    sems_g = (sem_g0, sem_g1, sem_g2, sem_g3)
    sems_s = (sem_s0, sem_s1, sem_s2, sem_s3)
    base = jnp.where(cid == 0, sid * GA0, GOFF + sid * GA1)
    NC = jnp.where(cid == 0, GA0 // CBL, GA1 // CBL)
    pltpu.sync_copy(sel_hbm.at[pl.ds(base * K, GA0 * K)], idx_v)

    def _gather_desc(cb, par):
        idx = idx_v.at[pl.ds(cb * CBL * K, CBL * K)]
        return pltpu.make_async_copy(
            h_hbm.at[idx], rows_v.at[par], sems_g[par])

    def _store_desc(cb, par):
        nb = base + cb * CBL
        return pltpu.make_async_copy(
            out_v.at[par], out_hbm.at[pl.ds(nb, CBL)], sems_s[par])

    for p in range(NBUF - 1):
        _gather_desc(p, p).start()

    @pl.loop(0, NC)
    def _chunk(cb):
        par_d = lax.rem(cb, NBUF)
        for par in range(NBUF):
            @pl.when(par_d == par)
            def _wait():
                _gather_desc(cb, par).wait()

            @pl.when((par_d == par) & (cb >= NBUF))
            def _drain_prev_store():
                _store_desc(cb - NBUF, par).wait()

        @pl.loop(0, CBL)
        def _node(i):
            r0 = i * K
            for k in range(D // 16):
                vals = [rows_v[par_d, r0 + r, pl.ds(16 * k, 16)]
                        for r in range(K)]
                out_v[par_d, i, pl.ds(16 * k, 16)] = (
                    _tree_sum(vals) * (1.0 / K))

        for par in range(NBUF):
            @pl.when(par_d == par)
            def _store():
                _store_desc(cb, par).start()

            @pl.when((par_d == par) & (cb + NBUF - 1 < NC))
            def _prefetch():
                _gather_desc(cb + NBUF - 1, (par + NBUF - 1) % NBUF).start()

    for p in range(NBUF):
        _store_desc(NC - NBUF + p, p).wait()


def _gather_mean(h, sel_flat):
    call = pl.kernel(
        _gmean_body,
        out_type=jax.ShapeDtypeStruct((NP, D), jnp.float32),
        mesh=_mesh,
        scratch_types=[
            pltpu.VMEM((GA0 * K,), jnp.int32),
            pltpu.VMEM((NBUF, CBL * K, D), jnp.float32),
            pltpu.VMEM((NBUF, CBL, D), jnp.float32),
            pltpu.SemaphoreType.DMA,
            pltpu.SemaphoreType.DMA,
            pltpu.SemaphoreType.DMA,
            pltpu.SemaphoreType.DMA,
            pltpu.SemaphoreType.DMA,
            pltpu.SemaphoreType.DMA,
            pltpu.SemaphoreType.DMA,
            pltpu.SemaphoreType.DMA,
        ],
        compiler_params=pltpu.CompilerParams(needs_layout_passes=False),
    )
    return call(h, sel_flat)


def _layer_body(h_ref, ng_ref, w_ref, b_ref, g_ref, be_ref, out_ref,
                bun_ref, s1_ref, s2_ref):
    p = pl.program_id(0)
    j = pl.program_id(1)

    @pl.when(p == 0)
    def _phase0():
        @pl.when(j == 0)
        def _init():
            s1_ref[...] = jnp.zeros_like(s1_ref)
            s2_ref[...] = jnp.zeros_like(s2_ref)

        hb = h_ref[...]
        bun = (jnp.dot(hb, w_ref[0:D, :], preferred_element_type=jnp.float32)
               + jnp.dot(ng_ref[...], w_ref[D:2 * D, :],
                         preferred_element_type=jnp.float32)
               + b_ref[...])
        nrm = jnp.sqrt(jnp.sum(bun * bun, axis=1, keepdims=True))
        bun = bun / (nrm + 1e-12)
        bun = jnp.maximum(bun, 0.0)
        row = j * BR + lax.broadcasted_iota(jnp.int32, (BR, 1), 0)
        m = (row < N).astype(jnp.float32)
        bm = bun * m
        s1_ref[...] += jnp.sum(bm, axis=0, keepdims=True)
        s2_ref[...] += jnp.sum(bm * bun, axis=0, keepdims=True)
        bun_ref[pl.ds(j * BR, BR), :] = bun

    @pl.when(p == 1)
    def _phase1():
        mu = s1_ref[...] * (1.0 / N)
        var = s2_ref[...] * (1.0 / N) - mu * mu
        den = jnp.sqrt(var + 1e-5)
        bun = bun_ref[pl.ds(j * BR, BR), :]
        out_ref[...] = h_ref[...] + g_ref[...] * (bun - mu) / den + be_ref[...]


def _sage_layer(h, neigh, w, b, gamma, beta):
    return pl.pallas_call(
        _layer_body,
        grid=(2, NB),
        in_specs=[
            pl.BlockSpec((BR, D), lambda p, j: (j, 0)),
            pl.BlockSpec((BR, D), lambda p, j: (j, 0)),
            pl.BlockSpec((2 * D, D), lambda p, j: (0, 0)),
            pl.BlockSpec((1, D), lambda p, j: (0, 0)),
            pl.BlockSpec((1, D), lambda p, j: (0, 0)),
            pl.BlockSpec((1, D), lambda p, j: (0, 0)),
        ],
        out_specs=pl.BlockSpec((BR, D), lambda p, j: (j, 0)),
        out_shape=jax.ShapeDtypeStruct((NP, D), jnp.float32),
        scratch_shapes=[
            pltpu.VMEM((NP, D), jnp.float32),
            pltpu.VMEM((1, D), jnp.float32),
            pltpu.VMEM((1, D), jnp.float32),
        ],
    )(h, neigh, w, b, gamma, beta)


def _read_body(h_ref, xx_ref, w1_ref, b1_ref, w2_ref, b2_ref, w3_ref, b3_ref,
               out_ref, tv_ref):
    h1 = jnp.maximum(
        jnp.dot(h_ref[...], w1_ref[...], preferred_element_type=jnp.float32)
        + b1_ref[...], 0.0)
    h2 = jnp.maximum(
        jnp.dot(h1, w2_ref[...], preferred_element_type=jnp.float32)
        + b2_ref[...], 0.0)
    out_ref[...] = (jnp.dot(h2, w3_ref[...], preferred_element_type=jnp.float32)
                    + b3_ref[...])
    xx = xx_ref[...]
    tv_ref[...] = jnp.log(xx + jnp.sqrt(xx * xx - 1.0))


def _readout(h, xx_s, w1, b1, w2, b2, w3p, b3p):
    return pl.pallas_call(
        _read_body,
        grid=(NB,),
        in_specs=[
            pl.BlockSpec((BR, D), lambda j: (j, 0)),
            pl.BlockSpec((BR, DEG), lambda j: (j, 0)),
            pl.BlockSpec((D, D // 2), lambda j: (0, 0)),
            pl.BlockSpec((1, D // 2), lambda j: (0, 0)),
            pl.BlockSpec((D // 2, D // 4), lambda j: (0, 0)),
            pl.BlockSpec((1, D // 4), lambda j: (0, 0)),
            pl.BlockSpec((D // 4, SIM_D), lambda j: (0, 0)),
            pl.BlockSpec((1, SIM_D), lambda j: (0, 0)),
        ],
        out_specs=[
            pl.BlockSpec((BR, SIM_D), lambda j: (j, 0)),
            pl.BlockSpec((BR, DEG), lambda j: (j, 0)),
        ],
        out_shape=[
            jax.ShapeDtypeStruct((NP, SIM_D), jnp.float32),
            jax.ShapeDtypeStruct((NP, DEG), jnp.float32),
        ],
    )(h, xx_s, w1, b1, w2, b2, w3p, b3p)


def kernel(x, params, edge_index):
    src = edge_index[0]
    x_pad = jnp.pad(x, ((0, NP - N), (0, 0)))
    src_pad = jnp.pad(src, (0, (NP - N + SA0 - SA1) * DEG))

    feat, qn2 = _project(x_pad, params['W_sim'], params['b_sim'][None, :])

    xx_f, ssrc_f = _sim_topk(feat, qn2.reshape(NP // SIM_D, SIM_D), src_pad)
    xx_s = xx_f.reshape(NP, DEG)
    sel_flat = jnp.pad(
        ssrc_f.reshape(NP, DEG)[:, :K].reshape(NP * K), (0, (GA0 - GA1) * K))

    h = x_pad
    for lp in params['layers']:
        neigh = _gather_mean(h, sel_flat)
        h = _sage_layer(h, neigh, lp['W'], lp['b'][None, :],
                        lp['gamma'][None, :], lp['beta'][None, :])

    w3p = jnp.pad(params['W_r3'], ((0, 0), (0, SIM_D - N_CLASSES)))
    b3p = jnp.pad(params['b_r3'], (0, SIM_D - N_CLASSES))[None, :]
    out_full, tv_full = _readout(h, xx_s, params['W_r1'],
                                 params['b_r1'][None, :], params['W_r2'],
                                 params['b_r2'][None, :], w3p, b3p)
    return out_full[:N, :N_CLASSES], tv_full[:N, :K]


# speedup vs baseline: 1.0998x; 1.0998x over previous
import jax
import jax.numpy as jnp
from jax import lax
from jax.experimental import pallas as pl
from jax.experimental.pallas import tpu as pltpu
from jax.experimental.pallas import tpu_sc as plsc

N = 10000
DEG = 16
D = 256
SIM_D = 128
K = 10
N_LAYERS = 3
N_CLASSES = 2
EPS = 1e-07

NW = 32
NP = 10240
SA0 = 576
SA1 = 64
SOFF = 16 * SA0
GA0 = 576
GA1 = 64
GOFF = 16 * GA0
CB = 8
CBL = 8
NBUF = 4
SW = 8
BR = 512
NB = NP // BR

_mesh = plsc.VectorSubcoreMesh(
    core_axis_name="c", subcore_axis_name="s", num_cores=2, num_subcores=16)


def _proj_body(x_ref, w_ref, b_ref, feat_ref, qn_ref):
    f = jnp.dot(x_ref[...], w_ref[...],
                preferred_element_type=jnp.float32) + b_ref[...]
    feat_ref[...] = f
    qn_ref[...] = jnp.sum(f * f, axis=1, keepdims=True)


def _project(x_pad, w_sim, b_sim):
    return pl.pallas_call(
        _proj_body,
        grid=(NB,),
        in_specs=[
            pl.BlockSpec((BR, D), lambda j: (j, 0)),
            pl.BlockSpec((D, SIM_D), lambda j: (0, 0)),
            pl.BlockSpec((1, SIM_D), lambda j: (0, 0)),
        ],
        out_specs=[
            pl.BlockSpec((BR, SIM_D), lambda j: (j, 0)),
            pl.BlockSpec((BR, 1), lambda j: (j, 0)),
        ],
        out_shape=[
            jax.ShapeDtypeStruct((NP, SIM_D), jnp.float32),
            jax.ShapeDtypeStruct((NP, 1), jnp.float32),
        ],
    )(x_pad, w_sim, b_sim)


def _tree_sum(vals):
    while len(vals) > 1:
        nxt = [vals[i] + vals[i + 1] for i in range(0, len(vals) - 1, 2)]
        if len(vals) % 2:
            nxt.append(vals[-1])
        vals = nxt
    return vals[0]


def _sim_body(feat_hbm, qn_hbm, src_hbm, xx_hbm, ssrc_hbm,
              qn_v, src_v, dstf_v, srcf_v, xx_v, ssrc_v,
              sem_g0, sem_g1, sem_g2, sem_g3, sem_s):
    cid = lax.axis_index("c")
    sid = lax.axis_index("s")
    sems = (sem_g0, sem_g1, sem_g2, sem_g3)
    base = jnp.where(cid == 0, sid * SA0, SOFF + sid * SA1)
    NC = jnp.where(cid == 0, SA0 // CB, SA1 // CB)

    def _make_descs(fsrc):
        def _descs(cb, par):
            idx = src_v.at[pl.ds(cb * CB * DEG, CB * DEG)]
            g = pltpu.make_async_copy(
                fsrc.at[idx], srcf_v.at[par], sems[par])
            d = pltpu.make_async_copy(
                fsrc.at[pl.ds(base + cb * CB, CB)], dstf_v.at[par],
                sems[par])
            return g, d
        return _descs

    def _xx_desc(cb):
        return pltpu.make_async_copy(
            xx_v.at[pl.ds(cb * CB * DEG, CB * DEG)],
            xx_hbm.at[pl.ds((base + cb * CB) * DEG, CB * DEG)], sem_s)

    def _ssrc_desc(cb):
        return pltpu.make_async_copy(
            ssrc_v.at[pl.ds(cb * CB * DEG, CB * DEG)],
            ssrc_hbm.at[pl.ds((base + cb * CB) * DEG, CB * DEG)], sem_s)

    def _chunk_body(cb, _descs, _fire):
        par_d = lax.rem(cb, NBUF)
        lb = cb * CB
        for par in range(NBUF):
            @pl.when(par_d == par)
            def _wait():
                g, d = _descs(cb, par)
                g.wait()
                d.wait()

        @pl.loop(0, CB)
        def _node(i):
            src_ids = src_v[pl.ds((lb + i) * DEG, DEG)]
            qs = plsc.load_gather(
                qn_v, [lax.shift_right_logical(src_ids, 7),
                       lax.bitwise_and(src_ids, 127)])
            dvec = [dstf_v[par_d, i, pl.ds(16 * k, 16)]
                    for k in range(SIM_D // 16)]
            qd = jnp.sum(_tree_sum([dv * dv for dv in dvec]))
            lane = lax.broadcasted_iota(jnp.int32, (DEG,), 0)
            dotv = jnp.zeros((DEG,), jnp.float32)
            for e in range(DEG):
                r = i * DEG + e
                prods = [srcf_v[par_d, r, pl.ds(16 * k, 16)] * dvec[k]
                         for k in range(SIM_D // 16)]
                dotv = jnp.where(lane == e,
                                 jnp.sum(_tree_sum(prods)), dotv)
            sqd = qd + qs - 2.0 * dotv
            xx = 1.0 + 2.0 * sqd / ((1.0 - qd) * (1.0 - qs)) + EPS
            sk, sv = plsc.sort_key_val(xx, src_ids, descending=True)
            xx_v[pl.ds((lb + i) * DEG, DEG)] = sk
            ssrc_v[pl.ds((lb + i) * DEG, DEG)] = sv

        _xx_desc(cb).start()
        _ssrc_desc(cb).start()

        @pl.when(cb >= SW)
        def _throttle_stores():
            _xx_desc(cb - SW).wait()
            _ssrc_desc(cb - SW).wait()

        for par in range(NBUF):
            @pl.when((par_d == par) & (cb + NBUF - 1 < NC))
            def _prefetch():
                _fire(cb + NBUF - 1, (par + NBUF - 1) % NBUF)

    pltpu.sync_copy(qn_hbm, qn_v)
    pltpu.sync_copy(src_hbm.at[pl.ds(base * DEG, SA0 * DEG)], src_v)

    def _pipeline(fsrc):
        _descs = _make_descs(fsrc)

        def _fire(cb, par):
            g, d = _descs(cb, par)
            g.start()
            d.start()

        for p in range(NBUF - 1):
            _fire(p, p)

        @pl.loop(0, NC)
        def _chunks(cb):
            _chunk_body(cb, _descs, _fire)

        @pl.loop(jnp.maximum(NC - SW, 0), NC)
        def _drain(cb):
            _xx_desc(cb).wait()
            _ssrc_desc(cb).wait()

    _pipeline(feat_hbm)


def _sim_topk(feat, qn, src_pad):
    call = pl.kernel(
        _sim_body,
        out_type=[
            jax.ShapeDtypeStruct((NP * DEG,), jnp.float32),
            jax.ShapeDtypeStruct((NP * DEG,), jnp.int32),
        ],
        mesh=_mesh,
        scratch_types=[
            pltpu.VMEM((NP // SIM_D, SIM_D), jnp.float32),
            pltpu.VMEM((SA0 * DEG,), jnp.int32),
            pltpu.VMEM((NBUF, CB, SIM_D), jnp.float32),
            pltpu.VMEM((NBUF, CB * DEG, SIM_D), jnp.float32),
            pltpu.VMEM((SA0 * DEG,), jnp.float32),
            pltpu.VMEM((SA0 * DEG,), jnp.int32),
            pltpu.SemaphoreType.DMA,
            pltpu.SemaphoreType.DMA,
            pltpu.SemaphoreType.DMA,
            pltpu.SemaphoreType.DMA,
            pltpu.SemaphoreType.DMA,
        ],
        compiler_params=pltpu.CompilerParams(needs_layout_passes=False),
    )
    return call(feat, qn, src_pad)


def _gmean_body(h_hbm, sel_hbm, out_hbm, idx_v, rows_v, out_v,
                sem_g0, sem_g1, sem_g2, sem_g3,
                sem_s0, sem_s1, sem_s2, sem_s3):
    cid = lax.axis_index("c")
    sid = lax.axis_index("s")
    sems_g = (sem_g0, sem_g1, sem_g2, sem_g3)
    sems_s = (sem_s0, sem_s1, sem_s2, sem_s3)
    base = jnp.where(cid == 0, sid * GA0, GOFF + sid * GA1)
    NC = jnp.where(cid == 0, GA0 // CBL, GA1 // CBL)
    pltpu.sync_copy(sel_hbm.at[pl.ds(base * K, GA0 * K)], idx_v)

    def _gather_desc(cb, par):
        idx = idx_v.at[pl.ds(cb * CBL * K, CBL * K)]
        return pltpu.make_async_copy(
            h_hbm.at[idx], rows_v.at[par], sems_g[par])

    def _store_desc(cb, par):
        nb = base + cb * CBL
        return pltpu.make_async_copy(
            out_v.at[par], out_hbm.at[pl.ds(nb, CBL)], sems_s[par])

    for p in range(NBUF - 1):
        _gather_desc(p, p).start()

    @pl.loop(0, NC)
    def _chunk(cb):
        par_d = lax.rem(cb, NBUF)
        for par in range(NBUF):
            @pl.when(par_d == par)
            def _wait():
                _gather_desc(cb, par).wait()

            @pl.when((par_d == par) & (cb >= NBUF))
            def _drain_prev_store():
                _store_desc(cb - NBUF, par).wait()

        @pl.loop(0, CBL)
        def _node(i):
            r0 = i * K
            for k in range(D // 16):
                vals = [rows_v[par_d, r0 + r, pl.ds(16 * k, 16)]
                        for r in range(K)]
                out_v[par_d, i, pl.ds(16 * k, 16)] = (
                    _tree_sum(vals) * (1.0 / K))

        for par in range(NBUF):
            @pl.when(par_d == par)
            def _store():
                _store_desc(cb, par).start()

            @pl.when((par_d == par) & (cb + NBUF - 1 < NC))
            def _prefetch():
                _gather_desc(cb + NBUF - 1, (par + NBUF - 1) % NBUF).start()

    for p in range(NBUF):
        _store_desc(NC - NBUF + p, p).wait()


def _gather_mean(h, sel_flat):
    call = pl.kernel(
        _gmean_body,
        out_type=jax.ShapeDtypeStruct((NP, D), jnp.float32),
        mesh=_mesh,
        scratch_types=[
            pltpu.VMEM((GA0 * K,), jnp.int32),
            pltpu.VMEM((NBUF, CBL * K, D), jnp.float32),
            pltpu.VMEM((NBUF, CBL, D), jnp.float32),
            pltpu.SemaphoreType.DMA,
            pltpu.SemaphoreType.DMA,
            pltpu.SemaphoreType.DMA,
            pltpu.SemaphoreType.DMA,
            pltpu.SemaphoreType.DMA,
            pltpu.SemaphoreType.DMA,
            pltpu.SemaphoreType.DMA,
            pltpu.SemaphoreType.DMA,
        ],
        compiler_params=pltpu.CompilerParams(needs_layout_passes=False),
    )
    return call(h, sel_flat)


def _layer_body(h_ref, ng_ref, w_ref, b_ref, g_ref, be_ref, out_ref,
                bun_ref, s1_ref, s2_ref):
    p = pl.program_id(0)
    j = pl.program_id(1)

    @pl.when(p == 0)
    def _phase0():
        @pl.when(j == 0)
        def _init():
            s1_ref[...] = jnp.zeros_like(s1_ref)
            s2_ref[...] = jnp.zeros_like(s2_ref)

        hb = h_ref[...]
        bun = (jnp.dot(hb, w_ref[0:D, :], preferred_element_type=jnp.float32)
               + jnp.dot(ng_ref[...], w_ref[D:2 * D, :],
                         preferred_element_type=jnp.float32)
               + b_ref[...])
        nrm = jnp.sqrt(jnp.sum(bun * bun, axis=1, keepdims=True))
        bun = bun / (nrm + 1e-12)
        bun = jnp.maximum(bun, 0.0)
        row = j * BR + lax.broadcasted_iota(jnp.int32, (BR, 1), 0)
        m = (row < N).astype(jnp.float32)
        bm = bun * m
        s1_ref[...] += jnp.sum(bm, axis=0, keepdims=True)
        s2_ref[...] += jnp.sum(bm * bun, axis=0, keepdims=True)
        bun_ref[pl.ds(j * BR, BR), :] = bun

    @pl.when(p == 1)
    def _phase1():
        mu = s1_ref[...] * (1.0 / N)
        var = s2_ref[...] * (1.0 / N) - mu * mu
        den = jnp.sqrt(var + 1e-5)
        bun = bun_ref[pl.ds(j * BR, BR), :]
        out_ref[...] = h_ref[...] + g_ref[...] * (bun - mu) / den + be_ref[...]


def _sage_layer(h, neigh, w, b, gamma, beta):
    return pl.pallas_call(
        _layer_body,
        grid=(2, NB),
        in_specs=[
            pl.BlockSpec((BR, D), lambda p, j: (j, 0)),
            pl.BlockSpec((BR, D), lambda p, j: (j, 0)),
            pl.BlockSpec((2 * D, D), lambda p, j: (0, 0)),
            pl.BlockSpec((1, D), lambda p, j: (0, 0)),
            pl.BlockSpec((1, D), lambda p, j: (0, 0)),
            pl.BlockSpec((1, D), lambda p, j: (0, 0)),
        ],
        out_specs=pl.BlockSpec((BR, D), lambda p, j: (j, 0)),
        out_shape=jax.ShapeDtypeStruct((NP, D), jnp.float32),
        scratch_shapes=[
            pltpu.VMEM((NP, D), jnp.float32),
            pltpu.VMEM((1, D), jnp.float32),
            pltpu.VMEM((1, D), jnp.float32),
        ],
    )(h, neigh, w, b, gamma, beta)


def _read_body(h_ref, xx_ref, w1_ref, b1_ref, w2_ref, b2_ref, w3_ref, b3_ref,
               out_ref, tv_ref):
    h1 = jnp.maximum(
        jnp.dot(h_ref[...], w1_ref[...], preferred_element_type=jnp.float32)
        + b1_ref[...], 0.0)
    h2 = jnp.maximum(
        jnp.dot(h1, w2_ref[...], preferred_element_type=jnp.float32)
        + b2_ref[...], 0.0)
    out_ref[...] = (jnp.dot(h2, w3_ref[...], preferred_element_type=jnp.float32)
                    + b3_ref[...])
    xx = xx_ref[...]
    tv_ref[...] = jnp.log(xx + jnp.sqrt(xx * xx - 1.0))


def _readout(h, xx_s, w1, b1, w2, b2, w3p, b3p):
    return pl.pallas_call(
        _read_body,
        grid=(NB,),
        in_specs=[
            pl.BlockSpec((BR, D), lambda j: (j, 0)),
            pl.BlockSpec((BR, DEG), lambda j: (j, 0)),
            pl.BlockSpec((D, D // 2), lambda j: (0, 0)),
            pl.BlockSpec((1, D // 2), lambda j: (0, 0)),
            pl.BlockSpec((D // 2, D // 4), lambda j: (0, 0)),
            pl.BlockSpec((1, D // 4), lambda j: (0, 0)),
            pl.BlockSpec((D // 4, SIM_D), lambda j: (0, 0)),
            pl.BlockSpec((1, SIM_D), lambda j: (0, 0)),
        ],
        out_specs=[
            pl.BlockSpec((BR, SIM_D), lambda j: (j, 0)),
            pl.BlockSpec((BR, DEG), lambda j: (j, 0)),
        ],
        out_shape=[
            jax.ShapeDtypeStruct((NP, SIM_D), jnp.float32),
            jax.ShapeDtypeStruct((NP, DEG), jnp.float32),
        ],
    )(h, xx_s, w1, b1, w2, b2, w3p, b3p)


def kernel(x, params, edge_index):
    src = edge_index[0]
    x_pad = jnp.pad(x, ((0, NP - N), (0, 0)))
    src_pad = jnp.pad(src, (0, (NP - N + SA0 - SA1) * DEG))

    feat, qn2 = _project(x_pad, params['W_sim'], params['b_sim'][None, :])

    xx_f, ssrc_f = _sim_topk(feat, qn2.reshape(NP // SIM_D, SIM_D), src_pad)
    xx_s = xx_f.reshape(NP, DEG)
    sel_flat = jnp.pad(
        ssrc_f.reshape(NP, DEG)[:, :K].reshape(NP * K), (0, (GA0 - GA1) * K))

    h = x_pad
    for lp in params['layers']:
        neigh = _gather_mean(h, sel_flat)
        h = _sage_layer(h, neigh, lp['W'], lp['b'][None, :],
                        lp['gamma'][None, :], lp['beta'][None, :])

    w3p = jnp.pad(params['W_r3'], ((0, 0), (0, SIM_D - N_CLASSES)))
    b3p = jnp.pad(params['b_r3'], (0, SIM_D - N_CLASSES))[None, :]
    out_full, tv_full = _readout(h, xx_s, params['W_r1'],
                                 params['b_r1'][None, :], params['W_r2'],
                                 params['b_r2'][None, :], w3p, b3p)
    return out_full[:N, :N_CLASSES], tv_full[:N, :K]


# speedup vs baseline: 1.1574x; 1.0524x over previous
import jax
import jax.numpy as jnp
from jax import lax
from jax.experimental import pallas as pl
from jax.experimental.pallas import tpu as pltpu
from jax.experimental.pallas import tpu_sc as plsc

N = 10000
DEG = 16
D = 256
SIM_D = 128
K = 10
N_LAYERS = 3
N_CLASSES = 2
EPS = 1e-07

NW = 32
NP = 10240
SA0 = 576
SA1 = 64
SOFF = 16 * SA0
GA0 = 576
GA1 = 64
GOFF = 16 * GA0
CB = 8
CBL = 8
NBUF = 4
SW = 8
BR = 1024
NB = NP // BR

_mesh = plsc.VectorSubcoreMesh(
    core_axis_name="c", subcore_axis_name="s", num_cores=2, num_subcores=16)


def _proj_body(x_ref, w_ref, b_ref, feat_ref, qn_ref):
    f = jnp.dot(x_ref[...], w_ref[...],
                preferred_element_type=jnp.float32) + b_ref[...]
    feat_ref[...] = f
    qn_ref[...] = jnp.sum(f * f, axis=1, keepdims=True)


def _project(x_pad, w_sim, b_sim):
    return pl.pallas_call(
        _proj_body,
        grid=(NB,),
        in_specs=[
            pl.BlockSpec((BR, D), lambda j: (j, 0)),
            pl.BlockSpec((D, SIM_D), lambda j: (0, 0)),
            pl.BlockSpec((1, SIM_D), lambda j: (0, 0)),
        ],
        out_specs=[
            pl.BlockSpec((BR, SIM_D), lambda j: (j, 0)),
            pl.BlockSpec((BR, 1), lambda j: (j, 0)),
        ],
        out_shape=[
            jax.ShapeDtypeStruct((NP, SIM_D), jnp.float32),
            jax.ShapeDtypeStruct((NP, 1), jnp.float32),
        ],
    )(x_pad, w_sim, b_sim)


def _tree_sum(vals):
    while len(vals) > 1:
        nxt = [vals[i] + vals[i + 1] for i in range(0, len(vals) - 1, 2)]
        if len(vals) % 2:
            nxt.append(vals[-1])
        vals = nxt
    return vals[0]


def _sim_body(feat_hbm, qn_hbm, src_hbm, xx_hbm, ssrc_hbm,
              qn_v, src_v, dstf_v, srcf_v, xx_v, ssrc_v,
              sem_g0, sem_g1, sem_g2, sem_g3, sem_s):
    cid = lax.axis_index("c")
    sid = lax.axis_index("s")
    sems = (sem_g0, sem_g1, sem_g2, sem_g3)
    base = jnp.where(cid == 0, sid * SA0, SOFF + sid * SA1)
    NC = jnp.where(cid == 0, SA0 // CB, SA1 // CB)

    def _make_descs(fsrc):
        def _descs(cb, par):
            idx = src_v.at[pl.ds(cb * CB * DEG, CB * DEG)]
            g = pltpu.make_async_copy(
                fsrc.at[idx], srcf_v.at[par], sems[par])
            d = pltpu.make_async_copy(
                fsrc.at[pl.ds(base + cb * CB, CB)], dstf_v.at[par],
                sems[par])
            return g, d
        return _descs

    def _xx_desc(cb):
        return pltpu.make_async_copy(
            xx_v.at[pl.ds(cb * CB * DEG, CB * DEG)],
            xx_hbm.at[pl.ds((base + cb * CB) * DEG, CB * DEG)], sem_s)

    def _ssrc_desc(cb):
        return pltpu.make_async_copy(
            ssrc_v.at[pl.ds(cb * CB * DEG, CB * DEG)],
            ssrc_hbm.at[pl.ds((base + cb * CB) * DEG, CB * DEG)], sem_s)

    def _chunk_body(cb, _descs, _fire):
        par_d = lax.rem(cb, NBUF)
        lb = cb * CB
        for par in range(NBUF):
            @pl.when(par_d == par)
            def _wait():
                g, d = _descs(cb, par)
                g.wait()
                d.wait()

        @pl.loop(0, CB)
        def _node(i):
            src_ids = src_v[pl.ds((lb + i) * DEG, DEG)]
            qs = plsc.load_gather(
                qn_v, [lax.shift_right_logical(src_ids, 7),
                       lax.bitwise_and(src_ids, 127)])
            dvec = [dstf_v[par_d, i, pl.ds(16 * k, 16)]
                    for k in range(SIM_D // 16)]
            qd = jnp.sum(_tree_sum([dv * dv for dv in dvec]))
            lane = lax.broadcasted_iota(jnp.int32, (DEG,), 0)
            dotv = jnp.zeros((DEG,), jnp.float32)
            for e in range(DEG):
                r = i * DEG + e
                prods = [srcf_v[par_d, r, pl.ds(16 * k, 16)] * dvec[k]
                         for k in range(SIM_D // 16)]
                dotv = jnp.where(lane == e,
                                 jnp.sum(_tree_sum(prods)), dotv)
            sqd = qd + qs - 2.0 * dotv
            xx = 1.0 + 2.0 * sqd / ((1.0 - qd) * (1.0 - qs)) + EPS
            sk, sv = plsc.sort_key_val(xx, src_ids, descending=True)
            xx_v[pl.ds((lb + i) * DEG, DEG)] = sk
            ssrc_v[pl.ds((lb + i) * DEG, DEG)] = sv

        _xx_desc(cb).start()
        _ssrc_desc(cb).start()

        @pl.when(cb >= SW)
        def _throttle_stores():
            _xx_desc(cb - SW).wait()
            _ssrc_desc(cb - SW).wait()

        for par in range(NBUF):
            @pl.when((par_d == par) & (cb + NBUF - 1 < NC))
            def _prefetch():
                _fire(cb + NBUF - 1, (par + NBUF - 1) % NBUF)

    pltpu.sync_copy(qn_hbm, qn_v)
    pltpu.sync_copy(src_hbm.at[pl.ds(base * DEG, SA0 * DEG)], src_v)

    def _pipeline(fsrc):
        _descs = _make_descs(fsrc)

        def _fire(cb, par):
            g, d = _descs(cb, par)
            g.start()
            d.start()

        for p in range(NBUF - 1):
            _fire(p, p)

        @pl.loop(0, NC)
        def _chunks(cb):
            _chunk_body(cb, _descs, _fire)

        @pl.loop(jnp.maximum(NC - SW, 0), NC)
        def _drain(cb):
            _xx_desc(cb).wait()
            _ssrc_desc(cb).wait()

    _pipeline(feat_hbm)


def _sim_topk(feat, qn, src_pad):
    call = pl.kernel(
        _sim_body,
        out_type=[
            jax.ShapeDtypeStruct((NP * DEG,), jnp.float32),
            jax.ShapeDtypeStruct((NP * DEG,), jnp.int32),
        ],
        mesh=_mesh,
        scratch_types=[
            pltpu.VMEM((NP // SIM_D, SIM_D), jnp.float32),
            pltpu.VMEM((SA0 * DEG,), jnp.int32),
            pltpu.VMEM((NBUF, CB, SIM_D), jnp.float32),
            pltpu.VMEM((NBUF, CB * DEG, SIM_D), jnp.float32),
            pltpu.VMEM((SA0 * DEG,), jnp.float32),
            pltpu.VMEM((SA0 * DEG,), jnp.int32),
            pltpu.SemaphoreType.DMA,
            pltpu.SemaphoreType.DMA,
            pltpu.SemaphoreType.DMA,
            pltpu.SemaphoreType.DMA,
            pltpu.SemaphoreType.DMA,
        ],
        compiler_params=pltpu.CompilerParams(needs_layout_passes=False),
    )
    return call(feat, qn, src_pad)


def _gmean_body(h_hbm, sel_hbm, out_hbm, idx_v, rows_v, out_v,
                sem_g0, sem_g1, sem_g2, sem_g3,
                sem_s0, sem_s1, sem_s2, sem_s3):
    cid = lax.axis_index("c")
    sid = lax.axis_index("s")
    sems_g = (sem_g0, sem_g1, sem_g2, sem_g3)
    sems_s = (sem_s0, sem_s1, sem_s2, sem_s3)
    base = jnp.where(cid == 0, sid * GA0, GOFF + sid * GA1)
    NC = jnp.where(cid == 0, GA0 // CBL, GA1 // CBL)
    pltpu.sync_copy(sel_hbm.at[pl.ds(base * K, GA0 * K)], idx_v)

    def _gather_desc(cb, par):
        idx = idx_v.at[pl.ds(cb * CBL * K, CBL * K)]
        return pltpu.make_async_copy(
            h_hbm.at[idx], rows_v.at[par], sems_g[par])

    def _store_desc(cb, par):
        nb = base + cb * CBL
        return pltpu.make_async_copy(
            out_v.at[par], out_hbm.at[pl.ds(nb, CBL)], sems_s[par])

    for p in range(NBUF - 1):
        _gather_desc(p, p).start()

    @pl.loop(0, NC)
    def _chunk(cb):
        par_d = lax.rem(cb, NBUF)
        for par in range(NBUF):
            @pl.when(par_d == par)
            def _wait():
                _gather_desc(cb, par).wait()

            @pl.when((par_d == par) & (cb >= NBUF))
            def _drain_prev_store():
                _store_desc(cb - NBUF, par).wait()

        @pl.loop(0, CBL)
        def _node(i):
            r0 = i * K
            for k in range(D // 16):
                vals = [rows_v[par_d, r0 + r, pl.ds(16 * k, 16)]
                        for r in range(K)]
                out_v[par_d, i, pl.ds(16 * k, 16)] = (
                    _tree_sum(vals) * (1.0 / K))

        for par in range(NBUF):
            @pl.when(par_d == par)
            def _store():
                _store_desc(cb, par).start()

            @pl.when((par_d == par) & (cb + NBUF - 1 < NC))
            def _prefetch():
                _gather_desc(cb + NBUF - 1, (par + NBUF - 1) % NBUF).start()

    for p in range(NBUF):
        _store_desc(NC - NBUF + p, p).wait()


def _gather_mean(h, sel_flat):
    call = pl.kernel(
        _gmean_body,
        out_type=jax.ShapeDtypeStruct((NP, D), jnp.float32),
        mesh=_mesh,
        scratch_types=[
            pltpu.VMEM((GA0 * K,), jnp.int32),
            pltpu.VMEM((NBUF, CBL * K, D), jnp.float32),
            pltpu.VMEM((NBUF, CBL, D), jnp.float32),
            pltpu.SemaphoreType.DMA,
            pltpu.SemaphoreType.DMA,
            pltpu.SemaphoreType.DMA,
            pltpu.SemaphoreType.DMA,
            pltpu.SemaphoreType.DMA,
            pltpu.SemaphoreType.DMA,
            pltpu.SemaphoreType.DMA,
            pltpu.SemaphoreType.DMA,
        ],
        compiler_params=pltpu.CompilerParams(needs_layout_passes=False),
    )
    return call(h, sel_flat)


def _layer_body(h_ref, ng_ref, w_ref, b_ref, g_ref, be_ref, out_ref,
                bun_ref, s1_ref, s2_ref):
    p = pl.program_id(0)
    j = pl.program_id(1)

    @pl.when(p == 0)
    def _phase0():
        @pl.when(j == 0)
        def _init():
            s1_ref[...] = jnp.zeros_like(s1_ref)
            s2_ref[...] = jnp.zeros_like(s2_ref)

        hb = h_ref[...]
        bun = (jnp.dot(hb, w_ref[0:D, :], preferred_element_type=jnp.float32)
               + jnp.dot(ng_ref[...], w_ref[D:2 * D, :],
                         preferred_element_type=jnp.float32)
               + b_ref[...])
        nrm = jnp.sqrt(jnp.sum(bun * bun, axis=1, keepdims=True))
        bun = bun / (nrm + 1e-12)
        bun = jnp.maximum(bun, 0.0)
        row = j * BR + lax.broadcasted_iota(jnp.int32, (BR, 1), 0)
        m = (row < N).astype(jnp.float32)
        bm = bun * m
        s1_ref[...] += jnp.sum(bm, axis=0, keepdims=True)
        s2_ref[...] += jnp.sum(bm * bun, axis=0, keepdims=True)
        bun_ref[pl.ds(j * BR, BR), :] = bun

    @pl.when(p == 1)
    def _phase1():
        mu = s1_ref[...] * (1.0 / N)
        var = s2_ref[...] * (1.0 / N) - mu * mu
        den = jnp.sqrt(var + 1e-5)
        bun = bun_ref[pl.ds(j * BR, BR), :]
        out_ref[...] = h_ref[...] + g_ref[...] * (bun - mu) / den + be_ref[...]


def _sage_layer(h, neigh, w, b, gamma, beta):
    return pl.pallas_call(
        _layer_body,
        grid=(2, NB),
        in_specs=[
            pl.BlockSpec((BR, D), lambda p, j: (j, 0)),
            pl.BlockSpec((BR, D), lambda p, j: (j, 0)),
            pl.BlockSpec((2 * D, D), lambda p, j: (0, 0)),
            pl.BlockSpec((1, D), lambda p, j: (0, 0)),
            pl.BlockSpec((1, D), lambda p, j: (0, 0)),
            pl.BlockSpec((1, D), lambda p, j: (0, 0)),
        ],
        out_specs=pl.BlockSpec((BR, D), lambda p, j: (j, 0)),
        out_shape=jax.ShapeDtypeStruct((NP, D), jnp.float32),
        scratch_shapes=[
            pltpu.VMEM((NP, D), jnp.float32),
            pltpu.VMEM((1, D), jnp.float32),
            pltpu.VMEM((1, D), jnp.float32),
        ],
    )(h, neigh, w, b, gamma, beta)


def _read_body(h_ref, xx_ref, w1_ref, b1_ref, w2_ref, b2_ref, w3_ref, b3_ref,
               out_ref, tv_ref):
    h1 = jnp.maximum(
        jnp.dot(h_ref[...], w1_ref[...], preferred_element_type=jnp.float32)
        + b1_ref[...], 0.0)
    h2 = jnp.maximum(
        jnp.dot(h1, w2_ref[...], preferred_element_type=jnp.float32)
        + b2_ref[...], 0.0)
    out_ref[...] = (jnp.dot(h2, w3_ref[...], preferred_element_type=jnp.float32)
                    + b3_ref[...])
    xx = xx_ref[...]
    tv_ref[...] = jnp.log(xx + jnp.sqrt(xx * xx - 1.0))


def _readout(h, xx_s, w1, b1, w2, b2, w3p, b3p):
    return pl.pallas_call(
        _read_body,
        grid=(NB,),
        in_specs=[
            pl.BlockSpec((BR, D), lambda j: (j, 0)),
            pl.BlockSpec((BR, DEG), lambda j: (j, 0)),
            pl.BlockSpec((D, D // 2), lambda j: (0, 0)),
            pl.BlockSpec((1, D // 2), lambda j: (0, 0)),
            pl.BlockSpec((D // 2, D // 4), lambda j: (0, 0)),
            pl.BlockSpec((1, D // 4), lambda j: (0, 0)),
            pl.BlockSpec((D // 4, SIM_D), lambda j: (0, 0)),
            pl.BlockSpec((1, SIM_D), lambda j: (0, 0)),
        ],
        out_specs=[
            pl.BlockSpec((BR, SIM_D), lambda j: (j, 0)),
            pl.BlockSpec((BR, DEG), lambda j: (j, 0)),
        ],
        out_shape=[
            jax.ShapeDtypeStruct((NP, SIM_D), jnp.float32),
            jax.ShapeDtypeStruct((NP, DEG), jnp.float32),
        ],
    )(h, xx_s, w1, b1, w2, b2, w3p, b3p)


def kernel(x, params, edge_index):
    src = edge_index[0]
    x_pad = jnp.pad(x, ((0, NP - N), (0, 0)))
    src_pad = jnp.pad(src, (0, (NP - N + SA0 - SA1) * DEG))

    feat, qn2 = _project(x_pad, params['W_sim'], params['b_sim'][None, :])

    xx_f, ssrc_f = _sim_topk(feat, qn2.reshape(NP // SIM_D, SIM_D), src_pad)
    xx_s = xx_f.reshape(NP, DEG)
    sel_flat = jnp.pad(
        ssrc_f.reshape(NP, DEG)[:, :K].reshape(NP * K), (0, (GA0 - GA1) * K))

    h = x_pad
    for lp in params['layers']:
        neigh = _gather_mean(h, sel_flat)
        h = _sage_layer(h, neigh, lp['W'], lp['b'][None, :],
                        lp['gamma'][None, :], lp['beta'][None, :])

    w3p = jnp.pad(params['W_r3'], ((0, 0), (0, SIM_D - N_CLASSES)))
    b3p = jnp.pad(params['b_r3'], (0, SIM_D - N_CLASSES))[None, :]
    out_full, tv_full = _readout(h, xx_s, params['W_r1'],
                                 params['b_r1'][None, :], params['W_r2'],
                                 params['b_r2'][None, :], w3p, b3p)
    return out_full[:N, :N_CLASSES], tv_full[:N, :K]
